# baseline (speedup 1.0000x reference)
import functools

import jax
import jax.numpy as jnp
from jax import lax
from jax.experimental import pallas as pl
from jax.experimental.pallas import tpu as pltpu
from jax.experimental.pallas import tpu_sc as plsc


@functools.lru_cache(maxsize=None)
def _make_sc_gather(V, D, B):
    info = plsc.get_sparse_core_info()
    NC, NS = info.num_cores, info.num_subcores
    NW = NC * NS
    assert B % NW == 0 and (B // NW) % 8 == 0
    b_per_w = B // NW
    mesh = plsc.VectorSubcoreMesh(core_axis_name="c", subcore_axis_name="s")

    @functools.partial(
        pl.kernel,
        mesh=mesh,
        out_type=jax.ShapeDtypeStruct((B, D), jnp.float32),
        scratch_types=[
            pltpu.VMEM((b_per_w,), jnp.int32),
            pltpu.VMEM((b_per_w, D), jnp.float32),
            pltpu.SemaphoreType.DMA,
        ],
    )
    def gather(table_hbm, idx_hbm, out_hbm, idx_v, rows_v, sem):
        wid = lax.axis_index("s") * NC + lax.axis_index("c")
        base = wid * b_per_w
        pltpu.sync_copy(idx_hbm.at[pl.ds(base, b_per_w)], idx_v)
        pltpu.async_copy(table_hbm.at[idx_v], rows_v, sem).wait()
        pltpu.sync_copy(rows_v, out_hbm.at[pl.ds(base, b_per_w)])

    return gather


def _mm_body(emb_ref, wu_ref, out_ref):
    out_ref[...] = jnp.dot(
        emb_ref[...], wu_ref[...], preferred_element_type=jnp.float32
    )


def _tc_matmul(emb, W_U, n_blk=512):
    M, K = emb.shape
    N = W_U.shape[1]
    return pl.pallas_call(
        _mm_body,
        grid=(pl.cdiv(N, n_blk),),
        in_specs=[
            pl.BlockSpec((M, K), lambda n: (0, 0)),
            pl.BlockSpec((K, n_blk), lambda n: (0, n)),
        ],
        out_specs=pl.BlockSpec((M, n_blk), lambda n: (0, n)),
        out_shape=jax.ShapeDtypeStruct((M, N), jnp.float32),
        compiler_params=pltpu.CompilerParams(
            dimension_semantics=("arbitrary",),
        ),
    )(emb, W_U)


def kernel(x, W_E, W_U):
    B, S = x.shape
    V, D = W_E.shape
    idx = x.reshape(-1).astype(jnp.int32)
    emb = _make_sc_gather(V, D, B * S)(W_E, idx)
    logits = _tc_matmul(emb, W_U)
    return logits.reshape(B, S, -1)
